# baseline (device time: 40202 ns/iter reference)
import jax
import jax.numpy as jnp
from jax import lax
from jax.experimental import pallas as pl
from jax.experimental.pallas import tpu as pltpu

M = 1024
N = 1024

ORDERS = (("x", "y", "z"), ("y", "z", "x"), ("z", "x", "y"))
CHAINS = (
    (0, 112, 0),
    (112, 112, 0),
    (224, 120, 0),
    (344, 112, 1),
    (456, 112, 1),
    (568, 120, 1),
    (688, 112, 2),
    (800, 112, 2),
    (912, 112, 2),
)
BAND_ROWS = ((0, 344), (344, 344), (688, 336))
RS_OFF = (0, 512, 768)
SEMS_PER_CHAIN = 7


def kernel(x, w_mat):
    def body(x_ref, w_ref, out_ref, c0, c1, c2, c3, c4, c5, c6, c7, c8,
             send_sems, recv_sems):
        comms = (c0, c1, c2, c3, c4, c5, c6, c7, c8)

        my = lax.axis_index("i")
        r4 = my % 4
        bz = my // 4
        bx = (r4 ^ (r4 >> 1)) & 1
        by = r4 // 2

        def pos_of(tx, ty, tz):
            return 4 * tz + 2 * ty + (tx ^ ty)

        partner = {
            "x": pos_of(1 - bx, by, bz),
            "y": pos_of(bx, 1 - by, bz),
            "z": pos_of(bx, by, 1 - bz),
        }
        bit = {"x": bx, "y": by, "z": bz}

        barrier_sem = pltpu.get_barrier_semaphore()
        for ax in ("x", "y", "z"):
            pl.semaphore_signal(
                barrier_sem, inc=1,
                device_id=(partner[ax],), device_id_type=pl.DeviceIdType.MESH,
            )

        def dscol(start, size):
            return pl.ds(pl.multiple_of(start, 128), size)

        cstart = [jnp.int32(0) for _ in CHAINS]

        def start_rs0(ci, sub):
            r0, rl, band = CHAINS[ci]
            a0, a1 = ORDERS[band][0], ORDERS[band][1]
            send_c = (1 - bit[a0]) * 512
            off_a = (1 - bit[a1]) * 256
            off = off_a if sub == 0 else 256 - off_a
            rdma = pltpu.make_async_remote_copy(
                src_ref=out_ref.at[pl.ds(r0, rl), dscol(send_c + off, 256)],
                dst_ref=comms[ci].at[:, dscol(RS_OFF[0] + off, 256)],
                send_sem=send_sems.at[ci * SEMS_PER_CHAIN + sub],
                recv_sem=recv_sems.at[ci * SEMS_PER_CHAIN + sub],
                device_id=(partner[a0],),
                device_id_type=pl.DeviceIdType.MESH,
            )
            rdma.start()
            return rdma

        def start_rs(ci, k):
            r0, rl, band = CHAINS[ci]
            half = 512 >> k
            ax = ORDERS[band][k]
            send_c = cstart[ci] + (1 - bit[ax]) * half
            rdma = pltpu.make_async_remote_copy(
                src_ref=out_ref.at[pl.ds(r0, rl), dscol(send_c, half)],
                dst_ref=comms[ci].at[:, pl.ds(RS_OFF[k], half)],
                send_sem=send_sems.at[ci * SEMS_PER_CHAIN + 1 + k],
                recv_sem=recv_sems.at[ci * SEMS_PER_CHAIN + 1 + k],
                device_id=(partner[ax],),
                device_id_type=pl.DeviceIdType.MESH,
            )
            rdma.start()
            return rdma

        def add_from_comm(ci, k, col, width, base):
            r0, rl, _ = CHAINS[ci]
            out_ref[pl.ds(r0, rl), dscol(col, width)] = (
                out_ref[pl.ds(r0, rl), dscol(col, width)]
                + comms[ci][:, dscol(RS_OFF[k] + (col - base), width)]
            )

        def finish_rs(ci, k):
            _, _, band = CHAINS[ci]
            half = 512 >> k
            keep = cstart[ci] + bit[ORDERS[band][k]] * half
            add_from_comm(ci, k, keep, half, keep)
            cstart[ci] = keep

        def start_leaf(ci):
            r0, rl, band = CHAINS[ci]
            ax = ORDERS[band][2]
            rdma = pltpu.make_async_remote_copy(
                src_ref=out_ref.at[pl.ds(r0, rl), dscol(cstart[ci], 256)],
                dst_ref=comms[ci].at[:, pl.ds(RS_OFF[2], 256)],
                send_sem=send_sems.at[ci * SEMS_PER_CHAIN + 3],
                recv_sem=recv_sems.at[ci * SEMS_PER_CHAIN + 3],
                device_id=(partner[ax],),
                device_id_type=pl.DeviceIdType.MESH,
            )
            rdma.start()
            return rdma

        def finish_leaf(ci):
            add_from_comm(ci, 2, cstart[ci], 256, cstart[ci])

        def start_ag(ci, ax, col, slot):
            r0, rl, _ = CHAINS[ci]
            rdma = pltpu.make_async_remote_copy(
                src_ref=out_ref.at[pl.ds(r0, rl), dscol(col, 256)],
                dst_ref=out_ref.at[pl.ds(r0, rl), dscol(col, 256)],
                send_sem=send_sems.at[ci * SEMS_PER_CHAIN + slot],
                recv_sem=recv_sems.at[ci * SEMS_PER_CHAIN + slot],
                device_id=(partner[ax],),
                device_id_type=pl.DeviceIdType.MESH,
            )
            rdma.start()
            return rdma

        rs0a = [None] * len(CHAINS)
        rs0b = [None] * len(CHAINS)
        rdmas = [None] * len(CHAINS)
        for band, (br0, brl) in enumerate(BAND_ROWS):
            b0 = bit[ORDERS[band][0]]
            send_c = (1 - b0) * 512
            out_ref[br0:br0 + brl, dscol(send_c, 512)] = jnp.dot(
                x_ref[br0:br0 + brl, :], w_ref[:, dscol(send_c, 512)],
                preferred_element_type=jnp.float32,
            )
            if band == 0:
                pl.semaphore_wait(barrier_sem, 3)
            for ci, (_, _, b) in enumerate(CHAINS):
                if b == band:
                    rs0a[ci] = start_rs0(ci, 0)
            for ci, (_, _, b) in enumerate(CHAINS):
                if b == band:
                    rs0b[ci] = start_rs0(ci, 1)
        for band, (br0, brl) in enumerate(BAND_ROWS):
            keep_c = bit[ORDERS[band][0]] * 512
            out_ref[br0:br0 + brl, dscol(keep_c, 512)] = jnp.dot(
                x_ref[br0:br0 + brl, :], w_ref[:, dscol(keep_c, 512)],
                preferred_element_type=jnp.float32,
            )

        ag_own = [None] * len(CHAINS)
        ag_fwd = [None] * len(CHAINS)
        for ci in range(len(CHAINS)):
            _, _, band = CHAINS[ci]
            keep0 = cstart[ci] + bit[ORDERS[band][0]] * 512
            ax1 = ORDERS[band][1]
            send1 = keep0 + (1 - bit[ax1]) * 256
            keep1 = keep0 + bit[ax1] * 256
            rs0a[ci].wait()
            add_from_comm(ci, 0, send1, 256, keep0)
            cstart[ci] = keep0
            rdmas[ci] = start_rs(ci, 1)
            rs0b[ci].wait()
            add_from_comm(ci, 0, keep1, 256, keep0)
        for ci in range(len(CHAINS)):
            rdmas[ci].wait()
            finish_rs(ci, 1)
            rdmas[ci] = start_leaf(ci)
        for ci in range(len(CHAINS)):
            _, _, band = CHAINS[ci]
            rdmas[ci].wait()
            finish_leaf(ci)
            rdmas[ci] = start_ag(ci, ORDERS[band][1], cstart[ci], 4)
            ag_own[ci] = start_ag(ci, ORDERS[band][0], cstart[ci], 5)
        for ci in range(len(CHAINS)):
            _, _, band = CHAINS[ci]
            rdmas[ci].wait()
            q1 = cstart[ci] + (1 - 2 * bit[ORDERS[band][1]]) * 256
            ag_fwd[ci] = start_ag(ci, ORDERS[band][0], q1, 6)
        for ci in range(len(CHAINS)):
            ag_own[ci].wait()
            ag_fwd[ci].wait()

    n_sems = SEMS_PER_CHAIN * len(CHAINS)
    return pl.pallas_call(
        body,
        out_shape=jax.ShapeDtypeStruct((M, N), jnp.float32),
        in_specs=[
            pl.BlockSpec(memory_space=pltpu.VMEM),
            pl.BlockSpec(memory_space=pltpu.VMEM),
        ],
        out_specs=pl.BlockSpec(memory_space=pltpu.VMEM),
        scratch_shapes=[
            pltpu.VMEM((rl, 1024), jnp.float32) for (_, rl, _) in CHAINS
        ] + [
            pltpu.SemaphoreType.DMA((n_sems,)),
            pltpu.SemaphoreType.DMA((n_sems,)),
        ],
        compiler_params=pltpu.CompilerParams(collective_id=0),
    )(x, w_mat)


# device time: 37122 ns/iter; 1.0830x vs baseline; 1.0830x over previous
import jax
import jax.numpy as jnp
from jax import lax
from jax.experimental import pallas as pl
from jax.experimental.pallas import tpu as pltpu

M = 1024
N = 1024

ORDERS = (("x", "y", "z"), ("y", "z", "x"), ("z", "x", "y"))
CHAINS = (
    (0, 112, 0),
    (112, 112, 0),
    (224, 120, 0),
    (344, 112, 1),
    (456, 112, 1),
    (568, 120, 1),
    (688, 112, 2),
    (800, 112, 2),
    (912, 112, 2),
)
BAND_ROWS = ((0, 344), (344, 344), (688, 336))
RS_OFF = (0, 512, 768)
SEMS_PER_CHAIN = 7


def kernel(x, w_mat):
    def body(x_ref, w_ref, out_ref, c0, c1, c2, c3, c4, c5, c6, c7, c8,
             send_sems, recv_sems):
        comms = (c0, c1, c2, c3, c4, c5, c6, c7, c8)

        my = lax.axis_index("i")
        r4 = my % 4
        bz = my // 4
        bx = (r4 ^ (r4 >> 1)) & 1
        by = r4 // 2

        def pos_of(tx, ty, tz):
            return 4 * tz + 2 * ty + (tx ^ ty)

        partner = {
            "x": pos_of(1 - bx, by, bz),
            "y": pos_of(bx, 1 - by, bz),
            "z": pos_of(bx, by, 1 - bz),
        }
        bit = {"x": bx, "y": by, "z": bz}

        barrier_sem = pltpu.get_barrier_semaphore()
        for ax in ("x", "y", "z"):
            pl.semaphore_signal(
                barrier_sem, inc=1,
                device_id=(partner[ax],), device_id_type=pl.DeviceIdType.MESH,
            )

        def dscol(start, size):
            return pl.ds(pl.multiple_of(start, 128), size)

        cstart = [jnp.int32(0) for _ in CHAINS]

        def start_rs0(ci, sub):
            r0, rl, band = CHAINS[ci]
            a0, a1 = ORDERS[band][0], ORDERS[band][1]
            send_c = (1 - bit[a0]) * 512
            off_a = (1 - bit[a1]) * 256
            off = off_a if sub == 0 else 256 - off_a
            rdma = pltpu.make_async_remote_copy(
                src_ref=out_ref.at[pl.ds(r0, rl), dscol(send_c + off, 256)],
                dst_ref=comms[ci].at[:, dscol(RS_OFF[0] + off, 256)],
                send_sem=send_sems.at[ci * SEMS_PER_CHAIN + sub],
                recv_sem=recv_sems.at[ci * SEMS_PER_CHAIN + sub],
                device_id=(partner[a0],),
                device_id_type=pl.DeviceIdType.MESH,
            )
            rdma.start()
            return rdma

        def start_rs(ci, k):
            r0, rl, band = CHAINS[ci]
            half = 512 >> k
            ax = ORDERS[band][k]
            send_c = cstart[ci] + (1 - bit[ax]) * half
            rdma = pltpu.make_async_remote_copy(
                src_ref=out_ref.at[pl.ds(r0, rl), dscol(send_c, half)],
                dst_ref=comms[ci].at[:, pl.ds(RS_OFF[k], half)],
                send_sem=send_sems.at[ci * SEMS_PER_CHAIN + 1 + k],
                recv_sem=recv_sems.at[ci * SEMS_PER_CHAIN + 1 + k],
                device_id=(partner[ax],),
                device_id_type=pl.DeviceIdType.MESH,
            )
            rdma.start()
            return rdma

        def add_from_comm(ci, k, col, width, base):
            r0, rl, _ = CHAINS[ci]
            out_ref[pl.ds(r0, rl), dscol(col, width)] = (
                out_ref[pl.ds(r0, rl), dscol(col, width)]
                + comms[ci][:, dscol(RS_OFF[k] + (col - base), width)]
            )

        def finish_rs(ci, k):
            _, _, band = CHAINS[ci]
            half = 512 >> k
            keep = cstart[ci] + bit[ORDERS[band][k]] * half
            add_from_comm(ci, k, keep, half, keep)
            cstart[ci] = keep

        def start_leaf(ci):
            r0, rl, band = CHAINS[ci]
            ax = ORDERS[band][2]
            rdma = pltpu.make_async_remote_copy(
                src_ref=out_ref.at[pl.ds(r0, rl), dscol(cstart[ci], 256)],
                dst_ref=comms[ci].at[:, pl.ds(RS_OFF[2], 256)],
                send_sem=send_sems.at[ci * SEMS_PER_CHAIN + 3],
                recv_sem=recv_sems.at[ci * SEMS_PER_CHAIN + 3],
                device_id=(partner[ax],),
                device_id_type=pl.DeviceIdType.MESH,
            )
            rdma.start()
            return rdma

        def finish_leaf(ci):
            add_from_comm(ci, 2, cstart[ci], 256, cstart[ci])

        def start_ag(ci, ax, col, slot):
            r0, rl, _ = CHAINS[ci]
            rdma = pltpu.make_async_remote_copy(
                src_ref=out_ref.at[pl.ds(r0, rl), dscol(col, 256)],
                dst_ref=out_ref.at[pl.ds(r0, rl), dscol(col, 256)],
                send_sem=send_sems.at[ci * SEMS_PER_CHAIN + slot],
                recv_sem=recv_sems.at[ci * SEMS_PER_CHAIN + slot],
                device_id=(partner[ax],),
                device_id_type=pl.DeviceIdType.MESH,
            )
            rdma.start()
            return rdma

        rs0a = [None] * len(CHAINS)
        rs0b = [None] * len(CHAINS)
        rdmas = [None] * len(CHAINS)
        for band, (br0, brl) in enumerate(BAND_ROWS):
            b0 = bit[ORDERS[band][0]]
            send_c = (1 - b0) * 512
            out_ref[br0:br0 + brl, dscol(send_c, 512)] = jnp.dot(
                x_ref[br0:br0 + brl, :], w_ref[:, dscol(send_c, 512)],
                preferred_element_type=jnp.float32,
            )
            if band == 0:
                pl.semaphore_wait(barrier_sem, 3)
            for ci, (_, _, b) in enumerate(CHAINS):
                if b == band:
                    rs0a[ci] = start_rs0(ci, 0)
            for ci, (_, _, b) in enumerate(CHAINS):
                if b == band:
                    rs0b[ci] = start_rs0(ci, 1)
        for band, (br0, brl) in enumerate(BAND_ROWS):
            keep_c = bit[ORDERS[band][0]] * 512
            out_ref[br0:br0 + brl, dscol(keep_c, 512)] = jnp.dot(
                x_ref[br0:br0 + brl, :], w_ref[:, dscol(keep_c, 512)],
                preferred_element_type=jnp.float32,
            )

        ORDER = (0, 3, 6, 1, 4, 7, 2, 5, 8)
        ag_own = [None] * len(CHAINS)
        ag_fwd = [None] * len(CHAINS)
        for ci in ORDER:
            _, _, band = CHAINS[ci]
            keep0 = cstart[ci] + bit[ORDERS[band][0]] * 512
            ax1 = ORDERS[band][1]
            send1 = keep0 + (1 - bit[ax1]) * 256
            keep1 = keep0 + bit[ax1] * 256
            rs0a[ci].wait()
            add_from_comm(ci, 0, send1, 256, keep0)
            cstart[ci] = keep0
            rdmas[ci] = start_rs(ci, 1)
            rs0b[ci].wait()
            add_from_comm(ci, 0, keep1, 256, keep0)
        for ci in ORDER:
            rdmas[ci].wait()
            finish_rs(ci, 1)
            rdmas[ci] = start_leaf(ci)
        for ci in ORDER:
            _, _, band = CHAINS[ci]
            rdmas[ci].wait()
            finish_leaf(ci)
            rdmas[ci] = start_ag(ci, ORDERS[band][1], cstart[ci], 4)
            ag_own[ci] = start_ag(ci, ORDERS[band][0], cstart[ci], 5)
        for ci in ORDER:
            _, _, band = CHAINS[ci]
            rdmas[ci].wait()
            q1 = cstart[ci] + (1 - 2 * bit[ORDERS[band][1]]) * 256
            ag_fwd[ci] = start_ag(ci, ORDERS[band][0], q1, 6)
        for ci in range(len(CHAINS)):
            ag_own[ci].wait()
            ag_fwd[ci].wait()

    n_sems = SEMS_PER_CHAIN * len(CHAINS)
    return pl.pallas_call(
        body,
        out_shape=jax.ShapeDtypeStruct((M, N), jnp.float32),
        in_specs=[
            pl.BlockSpec(memory_space=pltpu.VMEM),
            pl.BlockSpec(memory_space=pltpu.VMEM),
        ],
        out_specs=pl.BlockSpec(memory_space=pltpu.VMEM),
        scratch_shapes=[
            pltpu.VMEM((rl, 1024), jnp.float32) for (_, rl, _) in CHAINS
        ] + [
            pltpu.SemaphoreType.DMA((n_sems,)),
            pltpu.SemaphoreType.DMA((n_sems,)),
        ],
        compiler_params=pltpu.CompilerParams(collective_id=0),
    )(x, w_mat)
